# Initial kernel scaffold; baseline (speedup 1.0000x reference)
import jax
import jax.numpy as jnp
from jax.experimental import pallas as pl


def kernel(x, tables, W, b):
    raise NotImplementedError("write your pallas kernel here")


# speedup vs baseline: 13.0427x; 13.0427x over previous
import functools

import jax
import jax.numpy as jnp
from jax import lax
from jax.experimental import pallas as pl
from jax.experimental.pallas import tpu as pltpu
from jax.experimental.pallas import tpu_sc as plsc

F = 26
V = 1000
A = 128
B = 4096
FH = F // 2

_NC = 2
_NS = 16
_NW = _NC * _NS
_BPW = B // _NW
_CB = 8
_IDX_PER_CHUNK = _CB * FH
_NCHUNK = _BPW // _CB


def _make_proj_body(fold_bias):
    def body(t_ref, w_ref, b_ref, p_ref):
        t = t_ref[0]
        w = w_ref[:, 0, 0, :]
        p = lax.dot_general(
            t.astype(jnp.bfloat16), w.astype(jnp.bfloat16),
            (((1,), (1,)), ((), ())),
            preferred_element_type=jnp.float32,
        )
        if fold_bias:
            scale = jnp.where(pl.program_id(0) == 0, 1.0, 0.0)
            p = p + scale * b_ref[...]
        p_ref[0] = p
    return body


def _project(tables_half, W4, b2, f0, fold_bias):
    return pl.pallas_call(
        _make_proj_body(fold_bias),
        grid=(FH,),
        in_specs=[
            pl.BlockSpec((1, V, V), lambda f: (f, 0, 0)),
            pl.BlockSpec((A, 1, 1, V), lambda f, _f0=f0: (0, _f0 + f, 0, 0)),
            pl.BlockSpec((1, A), lambda f: (0, 0)),
        ],
        out_specs=pl.BlockSpec((1, V, A), lambda f: (f, 0, 0)),
        out_shape=jax.ShapeDtypeStruct((FH, V, A), jnp.float32),
    )(tables_half, W4, b2)


def _make_sc_body(final):
    def sc_body(pflat, xflat, *rest):
        if final:
            partial, out, idx_v, buf0, buf1, part_v, out_v, sem0, sem1 = rest
        else:
            out, idx_v, buf0, buf1, out_v, sem0, sem1 = rest
        wid = lax.axis_index("s") * _NC + lax.axis_index("c")
        base = wid * (_BPW * FH)
        pltpu.sync_copy(xflat.at[pl.ds(base, _BPW * FH)], idx_v)
        if final:
            pltpu.sync_copy(partial.at[pl.ds(wid * _BPW, _BPW)], part_v)

        def fire(ci, buf, sem):
            pltpu.async_copy(
                pflat.at[idx_v.at[pl.ds(ci * _IDX_PER_CHUNK, _IDX_PER_CHUNK)]],
                buf, sem)

        def drain(buf, sem):
            pltpu.make_async_copy(
                pflat.at[idx_v.at[pl.ds(0, _IDX_PER_CHUNK)]], buf, sem).wait()

        def compute(ci, buf):
            def row_body(r, _):
                rb = r * FH
                grow = ci * _CB + r
                acc = [buf[rb, pl.ds(j * 16, 16)] for j in range(8)]
                for f in range(1, FH):
                    for j in range(8):
                        acc[j] = acc[j] + buf[rb + f, pl.ds(j * 16, 16)]
                if final:
                    for j in range(8):
                        acc[j] = acc[j] + part_v[grow, pl.ds(j * 16, 16)]
                    lanes = lax.iota(jnp.int32, 16)
                    m = acc[0]
                    for j in range(1, 8):
                        m = jnp.maximum(m, acc[j])
                    for k in (8, 4, 2, 1):
                        m = jnp.maximum(
                            m, m.at[lanes ^ k].get(mode="promise_in_bounds"))
                    e = [jnp.exp(a - m) for a in acc]
                    s = e[0]
                    for j in range(1, 8):
                        s = s + e[j]
                    for k in (8, 4, 2, 1):
                        s = s + s.at[lanes ^ k].get(mode="promise_in_bounds")
                    for j in range(8):
                        out_v[grow, pl.ds(j * 16, 16)] = e[j] / s
                else:
                    for j in range(8):
                        out_v[grow, pl.ds(j * 16, 16)] = acc[j]
                return 0

            lax.fori_loop(0, _CB, row_body, 0)

        fire(0, buf0, sem0)

        def pair_body(i, _):
            c0 = 2 * i
            fire(c0 + 1, buf1, sem1)
            drain(buf0, sem0)
            compute(c0, buf0)

            @pl.when(c0 + 2 < _NCHUNK)
            def _():
                fire(c0 + 2, buf0, sem0)

            drain(buf1, sem1)
            compute(c0 + 1, buf1)
            return 0

        lax.fori_loop(0, _NCHUNK // 2, pair_body, 0)
        pltpu.sync_copy(out_v, out.at[pl.ds(wid * _BPW, _BPW)])

    return sc_body


def _sc_pass(pflat, xflat, partial):
    final = partial is not None
    mesh = plsc.VectorSubcoreMesh(core_axis_name="c", subcore_axis_name="s")
    scratch = [
        pltpu.VMEM((_BPW * FH,), jnp.int32),
        pltpu.VMEM((_IDX_PER_CHUNK, A), jnp.float32),
        pltpu.VMEM((_IDX_PER_CHUNK, A), jnp.float32),
    ]
    if final:
        scratch.append(pltpu.VMEM((_BPW, A), jnp.float32))
    scratch += [
        pltpu.VMEM((_BPW, A), jnp.float32),
        pltpu.SemaphoreType.DMA,
        pltpu.SemaphoreType.DMA,
    ]
    args = (pflat, xflat) + ((partial,) if final else ())
    return pl.kernel(
        _make_sc_body(final),
        out_type=jax.ShapeDtypeStruct((B, A), jnp.float32),
        mesh=mesh,
        scratch_types=scratch,
    )(*args)


def kernel(x, tables, W, b):
    W4 = W.reshape(A, F, 1, V)
    b2 = b.reshape(1, A)
    xi = x.astype(jnp.int32)
    offs = jnp.arange(FH, dtype=jnp.int32) * V
    p_lo = _project(tables[:FH], W4, b2, 0, True).reshape(FH * V, A)
    x_lo = (xi[:, :FH] + offs).reshape(-1)
    part = _sc_pass(p_lo, x_lo, None)
    p_hi = _project(tables[FH:], W4, b2, FH, False).reshape(FH * V, A)
    x_hi = (xi[:, FH:] + offs).reshape(-1)
    return _sc_pass(p_hi, x_hi, part)


# speedup vs baseline: 19.5763x; 1.5009x over previous
import functools

import jax
import jax.numpy as jnp
from jax import lax
from jax.experimental import pallas as pl
from jax.experimental.pallas import tpu as pltpu
from jax.experimental.pallas import tpu_sc as plsc

F = 26
V = 1000
A = 128
B = 4096
FH = F // 2

_NC = 2
_NS = 16
_NW = _NC * _NS
_BPW = B // _NW
_CB = 8
_IDX_PER_CHUNK = _CB * FH
_NCHUNK = _BPW // _CB


def _make_proj_body(fold_bias):
    def body(t_ref, w_ref, b_ref, p_ref):
        t = t_ref[0]
        w = w_ref[:, 0, 0, :]
        p = lax.dot_general(
            t.astype(jnp.bfloat16), w.astype(jnp.bfloat16),
            (((1,), (1,)), ((), ())),
            preferred_element_type=jnp.float32,
        )
        if fold_bias:
            scale = jnp.where(pl.program_id(0) == 0, 1.0, 0.0)
            p = p + scale * b_ref[...]
        p_ref[0] = p
    return body


def _project(tables, W4, b2, f0, fold_bias):
    return pl.pallas_call(
        _make_proj_body(fold_bias),
        grid=(FH,),
        in_specs=[
            pl.BlockSpec((1, V, V), lambda f, _f0=f0: (_f0 + f, 0, 0)),
            pl.BlockSpec((A, 1, 1, V), lambda f, _f0=f0: (0, _f0 + f, 0, 0)),
            pl.BlockSpec((1, A), lambda f: (0, 0)),
        ],
        out_specs=pl.BlockSpec((1, V, A), lambda f: (f, 0, 0)),
        out_shape=jax.ShapeDtypeStruct((FH, V, A), jnp.float32),
    )(tables, W4, b2)


def _make_sc_body(final):
    def sc_body(pflat, xflat, *rest):
        if final:
            partial, out, idx_v, buf0, buf1, part_v, out_v, sem0, sem1 = rest
        else:
            out, idx_v, buf0, buf1, out_v, sem0, sem1 = rest
        wid = lax.axis_index("s") * _NC + lax.axis_index("c")
        base = wid * (_BPW * FH)
        pltpu.sync_copy(xflat.at[pl.ds(base, _BPW * FH)], idx_v)
        if final:
            pltpu.sync_copy(partial.at[pl.ds(wid * _BPW, _BPW)], part_v)

        def fire(ci, buf, sem):
            pltpu.async_copy(
                pflat.at[idx_v.at[pl.ds(ci * _IDX_PER_CHUNK, _IDX_PER_CHUNK)]],
                buf, sem)

        def drain(buf, sem):
            pltpu.make_async_copy(
                pflat.at[idx_v.at[pl.ds(0, _IDX_PER_CHUNK)]], buf, sem).wait()

        def compute(ci, buf):
            def row_body(r, _):
                rb = r * FH
                grow = ci * _CB + r
                acc = [buf[rb, pl.ds(j * 16, 16)] for j in range(8)]
                for f in range(1, FH):
                    for j in range(8):
                        acc[j] = acc[j] + buf[rb + f, pl.ds(j * 16, 16)]
                if final:
                    for j in range(8):
                        acc[j] = acc[j] + part_v[grow, pl.ds(j * 16, 16)]
                    lanes = lax.iota(jnp.int32, 16)
                    m = acc[0]
                    for j in range(1, 8):
                        m = jnp.maximum(m, acc[j])
                    for k in (8, 4, 2, 1):
                        m = jnp.maximum(
                            m, m.at[lanes ^ k].get(mode="promise_in_bounds"))
                    e = [jnp.exp(a - m) for a in acc]
                    s = e[0]
                    for j in range(1, 8):
                        s = s + e[j]
                    for k in (8, 4, 2, 1):
                        s = s + s.at[lanes ^ k].get(mode="promise_in_bounds")
                    for j in range(8):
                        out_v[grow, pl.ds(j * 16, 16)] = e[j] / s
                else:
                    for j in range(8):
                        out_v[grow, pl.ds(j * 16, 16)] = acc[j]
                return 0

            lax.fori_loop(0, _CB, row_body, 0)

        fire(0, buf0, sem0)

        def pair_body(i, _):
            c0 = 2 * i
            fire(c0 + 1, buf1, sem1)
            drain(buf0, sem0)
            compute(c0, buf0)

            @pl.when(c0 + 2 < _NCHUNK)
            def _():
                fire(c0 + 2, buf0, sem0)

            drain(buf1, sem1)
            compute(c0 + 1, buf1)
            return 0

        lax.fori_loop(0, _NCHUNK // 2, pair_body, 0)
        pltpu.sync_copy(out_v, out.at[pl.ds(wid * _BPW, _BPW)])

    return sc_body


def _sc_pass(pflat, xflat, partial):
    final = partial is not None
    mesh = plsc.VectorSubcoreMesh(core_axis_name="c", subcore_axis_name="s")
    scratch = [
        pltpu.VMEM((_BPW * FH,), jnp.int32),
        pltpu.VMEM((_IDX_PER_CHUNK, A), jnp.float32),
        pltpu.VMEM((_IDX_PER_CHUNK, A), jnp.float32),
    ]
    if final:
        scratch.append(pltpu.VMEM((_BPW, A), jnp.float32))
    scratch += [
        pltpu.VMEM((_BPW, A), jnp.float32),
        pltpu.SemaphoreType.DMA,
        pltpu.SemaphoreType.DMA,
    ]
    args = (pflat, xflat) + ((partial,) if final else ())
    return pl.kernel(
        _make_sc_body(final),
        out_type=jax.ShapeDtypeStruct((B, A), jnp.float32),
        mesh=mesh,
        scratch_types=scratch,
    )(*args)


def kernel(x, tables, W, b):
    W4 = W.reshape(A, F, 1, V)
    b2 = b.reshape(1, A)
    xi = x.astype(jnp.int32)
    offs = jnp.arange(FH, dtype=jnp.int32) * V
    p_lo = _project(tables, W4, b2, 0, True).reshape(FH * V, A)
    x_lo = (xi[:, :FH] + offs).reshape(-1)
    part = _sc_pass(p_lo, x_lo, None)
    p_hi = _project(tables, W4, b2, FH, False).reshape(FH * V, A)
    x_hi = (xi[:, FH:] + offs).reshape(-1)
    return _sc_pass(p_hi, x_hi, part)


# speedup vs baseline: 26.3291x; 1.3449x over previous
import functools

import jax
import jax.numpy as jnp
from jax import lax
from jax.experimental import pallas as pl
from jax.experimental.pallas import tpu as pltpu
from jax.experimental.pallas import tpu_sc as plsc

F = 26
V = 1000
A = 128
B = 4096
FH = F // 2

_NC = 2
_NS = 16
_NW = _NC * _NS
_BPW = B // _NW
_CB = 8
_IDX_PER_CHUNK = _CB * FH
_NCHUNK = _BPW // _CB


def _make_proj_body(fold_bias):
    def body(t_ref, w_ref, b_ref, p_ref):
        t = t_ref[0]
        w = w_ref[0]
        p = lax.dot_general(
            t.astype(jnp.bfloat16), w.astype(jnp.bfloat16),
            (((1,), (0,)), ((), ())),
            preferred_element_type=jnp.float32,
        )
        if fold_bias:
            scale = jnp.where(pl.program_id(0) == 0, 1.0, 0.0)
            p = p + scale * b_ref[...]
        p_ref[0] = p
    return body


def _project(tables, Wt3, b2, f0, fold_bias):
    return pl.pallas_call(
        _make_proj_body(fold_bias),
        grid=(FH,),
        in_specs=[
            pl.BlockSpec((1, V, V), lambda f, _f0=f0: (_f0 + f, 0, 0)),
            pl.BlockSpec((1, V, A), lambda f, _f0=f0: (_f0 + f, 0, 0)),
            pl.BlockSpec((1, A), lambda f: (0, 0)),
        ],
        out_specs=pl.BlockSpec((1, V, A), lambda f: (f, 0, 0)),
        out_shape=jax.ShapeDtypeStruct((FH, V, A), jnp.float32),
    )(tables, Wt3, b2)


def _make_sc_body(final):
    def sc_body(pflat, xflat, *rest):
        if final:
            partial, out, idx_v, buf0, buf1, part_v, out_v, sem0, sem1 = rest
        else:
            out, idx_v, buf0, buf1, out_v, sem0, sem1 = rest
        wid = lax.axis_index("s") * _NC + lax.axis_index("c")
        base = wid * (_BPW * FH)
        pltpu.sync_copy(xflat.at[pl.ds(base, _BPW * FH)], idx_v)
        if final:
            pltpu.sync_copy(partial.at[pl.ds(wid * _BPW, _BPW)], part_v)

        def fire(ci, buf, sem):
            pltpu.async_copy(
                pflat.at[idx_v.at[pl.ds(ci * _IDX_PER_CHUNK, _IDX_PER_CHUNK)]],
                buf, sem)

        def drain(buf, sem):
            pltpu.make_async_copy(
                pflat.at[idx_v.at[pl.ds(0, _IDX_PER_CHUNK)]], buf, sem).wait()

        def compute(ci, buf):
            def row_body(r, _):
                rb = r * FH
                grow = ci * _CB + r
                acc = [buf[rb, pl.ds(j * 16, 16)] for j in range(8)]
                for f in range(1, FH):
                    for j in range(8):
                        acc[j] = acc[j] + buf[rb + f, pl.ds(j * 16, 16)]
                if final:
                    for j in range(8):
                        acc[j] = acc[j] + part_v[grow, pl.ds(j * 16, 16)]
                    lanes = lax.iota(jnp.int32, 16)
                    m = acc[0]
                    for j in range(1, 8):
                        m = jnp.maximum(m, acc[j])
                    for k in (8, 4, 2, 1):
                        m = jnp.maximum(
                            m, m.at[lanes ^ k].get(mode="promise_in_bounds"))
                    e = [jnp.exp(a - m) for a in acc]
                    s = e[0]
                    for j in range(1, 8):
                        s = s + e[j]
                    for k in (8, 4, 2, 1):
                        s = s + s.at[lanes ^ k].get(mode="promise_in_bounds")
                    for j in range(8):
                        out_v[grow, pl.ds(j * 16, 16)] = e[j] / s
                else:
                    for j in range(8):
                        out_v[grow, pl.ds(j * 16, 16)] = acc[j]
                return 0

            lax.fori_loop(0, _CB, row_body, 0)

        fire(0, buf0, sem0)

        def pair_body(i, _):
            c0 = 2 * i
            fire(c0 + 1, buf1, sem1)
            drain(buf0, sem0)
            compute(c0, buf0)

            @pl.when(c0 + 2 < _NCHUNK)
            def _():
                fire(c0 + 2, buf0, sem0)

            drain(buf1, sem1)
            compute(c0 + 1, buf1)
            return 0

        lax.fori_loop(0, _NCHUNK // 2, pair_body, 0)
        pltpu.sync_copy(out_v, out.at[pl.ds(wid * _BPW, _BPW)])

    return sc_body


def _sc_pass(pflat, xflat, partial):
    final = partial is not None
    mesh = plsc.VectorSubcoreMesh(core_axis_name="c", subcore_axis_name="s")
    scratch = [
        pltpu.VMEM((_BPW * FH,), jnp.int32),
        pltpu.VMEM((_IDX_PER_CHUNK, A), jnp.float32),
        pltpu.VMEM((_IDX_PER_CHUNK, A), jnp.float32),
    ]
    if final:
        scratch.append(pltpu.VMEM((_BPW, A), jnp.float32))
    scratch += [
        pltpu.VMEM((_BPW, A), jnp.float32),
        pltpu.SemaphoreType.DMA,
        pltpu.SemaphoreType.DMA,
    ]
    args = (pflat, xflat) + ((partial,) if final else ())
    return pl.kernel(
        _make_sc_body(final),
        out_type=jax.ShapeDtypeStruct((B, A), jnp.float32),
        mesh=mesh,
        scratch_types=scratch,
    )(*args)


def kernel(x, tables, W, b):
    Wt3 = W.T.reshape(F, V, A)
    b2 = b.reshape(1, A)
    xi = x.astype(jnp.int32)
    offs = jnp.arange(FH, dtype=jnp.int32) * V
    p_lo = _project(tables, Wt3, b2, 0, True).reshape(FH * V, A)
    x_lo = (xi[:, :FH] + offs).reshape(-1)
    part = _sc_pass(p_lo, x_lo, None)
    p_hi = _project(tables, Wt3, b2, FH, False).reshape(FH * V, A)
    x_hi = (xi[:, FH:] + offs).reshape(-1)
    return _sc_pass(p_hi, x_hi, part)
